# baseline (speedup 1.0000x reference)
import functools

import jax
import jax.numpy as jnp
from jax import lax
from jax.experimental import pallas as pl
from jax.experimental.pallas import tpu as pltpu
from jax.experimental.pallas import tpu_sc as plsc

_NC = 2
_NS = 16
_L = 16


def _sc_body(x_hbm, m_hbm, o_hbm, mbuf, xbuf, *, rows_per_w, hw):
    wid = lax.axis_index("s") * _NC + lax.axis_index("c")
    base = wid * rows_per_w
    pltpu.sync_copy(m_hbm.at[wid], mbuf)

    def row_body(r, carry):
        pltpu.sync_copy(x_hbm.at[base + r], xbuf)

        def mul_body(i, c):
            o = i * _L
            xbuf[pl.ds(o, _L)] = xbuf[pl.ds(o, _L)] * mbuf[pl.ds(o, _L)]
            return c

        lax.fori_loop(0, hw // _L, mul_body, 0, unroll=8)
        pltpu.sync_copy(xbuf, o_hbm.at[base + r])
        return carry

    lax.fori_loop(0, rows_per_w, row_body, 0)


def kernel(input, mask):
    B, C, K, H, W = input.shape
    BC = B * C
    HW = H * W
    NW = _NC * _NS
    rows_per_w = (BC * K) // NW

    x = input.reshape(BC * K, HW)
    m = mask.reshape(BC, HW)

    mesh = plsc.VectorSubcoreMesh(core_axis_name="c", subcore_axis_name="s")
    run = functools.partial(
        pl.kernel,
        mesh=mesh,
        out_type=jax.ShapeDtypeStruct((BC * K, HW), x.dtype),
        scratch_types=[
            pltpu.VMEM((HW,), x.dtype),
            pltpu.VMEM((HW,), x.dtype),
        ],
    )(functools.partial(_sc_body, rows_per_w=rows_per_w, hw=HW))
    out = run(x, m)
    return out.reshape(B, C, K, H, W)


# speedup vs baseline: 7.2742x; 7.2742x over previous
import jax
import jax.numpy as jnp
from jax.experimental import pallas as pl
from jax.experimental.pallas import tpu as pltpu


def _body(m_ref, x_ref, o_ref):
    o_ref[...] = x_ref[...] * m_ref[...]


def kernel(input, mask):
    B, C, K, H, W = input.shape
    BC = B * C
    x = input.reshape(BC * K, H, W)
    m = mask.reshape(BC, H, W)

    ROWS = 16
    n = (BC * K) // ROWS

    out = pl.pallas_call(
        _body,
        grid=(n,),
        in_specs=[
            pl.BlockSpec((1, H, W), lambda j: (j * ROWS // K, 0, 0)),
            pl.BlockSpec((ROWS, H, W), lambda j: (j, 0, 0)),
        ],
        out_specs=pl.BlockSpec((ROWS, H, W), lambda j: (j, 0, 0)),
        out_shape=jax.ShapeDtypeStruct((BC * K, H, W), x.dtype),
        compiler_params=pltpu.CompilerParams(
            dimension_semantics=("arbitrary",),
        ),
    )(m, x)
    return out.reshape(B, C, K, H, W)


# speedup vs baseline: 7.3841x; 1.0151x over previous
import jax
import jax.numpy as jnp
from jax.experimental import pallas as pl
from jax.experimental.pallas import tpu as pltpu


def _body(m_ref, x_ref, o_ref):
    g, h, w = x_ref.shape
    mg = m_ref.shape[0]
    x = x_ref[...].reshape(mg, g // mg, h, w)
    o_ref[...] = (x * m_ref[...][:, None]).reshape(g, h, w)


def kernel(input, mask):
    B, C, K, H, W = input.shape
    BC = B * C
    x = input.reshape(BC * K, H, W)
    m = mask.reshape(BC, H, W)

    ROWS = 64
    n = (BC * K) // ROWS

    out = pl.pallas_call(
        _body,
        grid=(n,),
        in_specs=[
            pl.BlockSpec((ROWS // K, H, W), lambda j: (j, 0, 0)),
            pl.BlockSpec((ROWS, H, W), lambda j: (j, 0, 0)),
        ],
        out_specs=pl.BlockSpec((ROWS, H, W), lambda j: (j, 0, 0)),
        out_shape=jax.ShapeDtypeStruct((BC * K, H, W), x.dtype),
        compiler_params=pltpu.CompilerParams(
            dimension_semantics=("arbitrary",),
            vmem_limit_bytes=110 * 1024 * 1024,
        ),
    )(m, x)
    return out.reshape(B, C, K, H, W)
